# baseline (speedup 1.0000x reference)
import functools

import jax
import jax.numpy as jnp
from jax.experimental import pallas as pl

_DN_T = (((0,), (0,)), ((), ()))
_DN_M = (((1,), (0,)), ((), ()))


def _split(v):
    hi = v.astype(jnp.bfloat16)
    lo = (v - hi.astype(jnp.float32)).astype(jnp.bfloat16)
    return hi, lo


def _dot3(a_hi, a_lo, b_hi, b_lo, dn):
    f = functools.partial(jax.lax.dot_general, dimension_numbers=dn,
                          preferred_element_type=jnp.float32)
    return f(a_hi, b_hi) + f(a_hi, b_lo) + f(a_lo, b_hi)


def _fdot(a, b):
    return jax.lax.dot_general(a, b, _DN_M, preferred_element_type=jnp.float32)


_G = 4


def _gnn_body(a_ref, x_ref,
              wn0_ref, bn0_ref, wnb0_ref, bnb0_ref,
              wn1_ref, bn1_ref, wnb1_ref, bnb1_ref,
              wn2_ref, bn2_ref, wnb2_ref, bnb2_ref,
              nodes_out_ref, glob_out_ref):
    n = a_ref.shape[1]
    ii = jax.lax.broadcasted_iota(jnp.int32, (n, n), 0)
    jj = jax.lax.broadcasted_iota(jnp.int32, (n, n), 1)
    t_lt = (jj < ii).astype(jnp.bfloat16)
    t_le = (ii <= jj).astype(jnp.bfloat16)
    ones_col = jnp.ones((n, 1), jnp.bfloat16)

    layers = ((wn0_ref, bn0_ref, wnb0_ref, bnb0_ref),
              (wn1_ref, bn1_ref, wnb1_ref, bnb1_ref),
              (wn2_ref, bn2_ref, wnb2_ref, bnb2_ref))
    w_splits = [(_split(wn[...]), _split(wnb[...])) for (wn, _, wnb, _) in layers]
    fdot_t = functools.partial(jax.lax.dot_general, dimension_numbers=_DN_T,
                               preferred_element_type=jnp.float32)

    rowcnts = [jnp.sum(a_ref[g], axis=1, keepdims=True, dtype=jnp.float32)
               for g in range(_G)]
    rowcnt_all = jnp.concatenate(rowcnts, axis=1)
    ra_hi, ra_lo = _split(rowcnt_all)
    rstart_all = _fdot(t_lt, ra_hi) + _fdot(t_lt, ra_lo)
    colcnts = [jnp.sum(a_ref[g], axis=0, keepdims=True, dtype=jnp.float32)
               for g in range(_G)]
    colcnt_all = jnp.concatenate(colcnts, axis=0)
    ca_hi, ca_lo = _split(colcnt_all)
    cum_all = _fdot(ca_hi, t_le) + _fdot(ca_lo, t_le)

    w_his, w_los, presents, xs = [], [], [], []
    for g in range(_G):
        rstart = rstart_all[:, g:g + 1]
        rend = rstart + rowcnts[g]
        cum = cum_all[g:g + 1, :]
        cumprev = cum - colcnts[g]
        w_t = jnp.maximum(jnp.minimum(rend, cum) - jnp.maximum(rstart, cumprev), 0.0)
        w_hi, w_lo = _split(w_t)
        w_his.append(w_hi)
        w_los.append(w_lo)

        colcnt_col = jax.lax.dot_general(
            a_ref[g], ones_col, _DN_T, preferred_element_type=jnp.float32)
        presents.append(colcnt_col > 0.0)
        xs.append(x_ref[g])

    for i, (wn, bn, wnb, bnb) in enumerate(layers):
        (wn_hi, wn_lo), (wnb_hi, wnb_lo) = w_splits[i]
        sp = [_split(x) for x in xs]
        pools = [fdot_t(w_his[g], sp[g][0]) + fdot_t(w_his[g], sp[g][1])
                 + fdot_t(w_los[g], sp[g][0]) for g in range(_G)]
        psp = [_split(p) for p in pools]
        h_nodes = [jnp.maximum(_dot3(sp[g][0], sp[g][1], wn_hi, wn_lo, _DN_M)
                               + bn[...], 0.0) for g in range(_G)]
        h_nbs = [jnp.maximum(_dot3(psp[g][0], psp[g][1], wnb_hi, wnb_lo, _DN_M)
                             + bnb[...], 0.0) for g in range(_G)]
        if i == len(layers) - 1:
            for g in range(_G):
                glob_out_ref[g] = jnp.sum(xs[g], axis=0, keepdims=True)
        xs = [jnp.where(presents[g], h_nbs[g], h_nodes[g]) for g in range(_G)]
    for g in range(_G):
        nodes_out_ref[g] = xs[g]


def kernel(nodes, adjacent,
           W_node_0, b_node_0, W_nb_0, b_nb_0,
           W_node_1, b_node_1, W_nb_1, b_nb_1,
           W_node_2, b_node_2, W_nb_2, b_nb_2):
    B, N, D = nodes.shape
    w_spec = pl.BlockSpec((D, D), lambda b: (0, 0))
    bias_spec = pl.BlockSpec((1, D), lambda b: (0, 0))
    nodes_out, glob = pl.pallas_call(
        _gnn_body,
        grid=(B // _G,),
        in_specs=[
            pl.BlockSpec((_G, N, N), lambda b: (b, 0, 0)),
            pl.BlockSpec((_G, N, D), lambda b: (b, 0, 0)),
            w_spec, bias_spec, w_spec, bias_spec,
            w_spec, bias_spec, w_spec, bias_spec,
            w_spec, bias_spec, w_spec, bias_spec,
        ],
        out_specs=[
            pl.BlockSpec((_G, N, D), lambda b: (b, 0, 0)),
            pl.BlockSpec((_G, 1, D), lambda b: (b, 0, 0)),
        ],
        out_shape=[
            jax.ShapeDtypeStruct((B, N, D), jnp.float32),
            jax.ShapeDtypeStruct((B, 1, D), jnp.float32),
        ],
    )(adjacent.astype(jnp.bfloat16), nodes,
      W_node_0, b_node_0.reshape(1, D), W_nb_0, b_nb_0.reshape(1, D),
      W_node_1, b_node_1.reshape(1, D), W_nb_1, b_nb_1.reshape(1, D),
      W_node_2, b_node_2.reshape(1, D), W_nb_2, b_nb_2.reshape(1, D))
    return (nodes_out, glob.reshape(B, D))


# speedup vs baseline: 1.1743x; 1.1743x over previous
import functools

import jax
import jax.numpy as jnp
from jax.experimental import pallas as pl

_DN_T = (((0,), (0,)), ((), ()))
_DN_M = (((1,), (0,)), ((), ()))


def _split(v):
    hi = v.astype(jnp.bfloat16)
    lo = (v - hi.astype(jnp.float32)).astype(jnp.bfloat16)
    return hi, lo


def _dot3(a_hi, a_lo, b_hi, b_lo, dn):
    f = functools.partial(jax.lax.dot_general, dimension_numbers=dn,
                          preferred_element_type=jnp.float32)
    return f(a_hi, b_hi) + f(a_hi, b_lo) + f(a_lo, b_hi)


def _fdot(a, b):
    return jax.lax.dot_general(a, b, _DN_M, preferred_element_type=jnp.float32)


_G = 4


def _gnn_body(a_ref, x_ref,
              wn0_ref, bn0_ref, wnb0_ref, bnb0_ref,
              wn1_ref, bn1_ref, wnb1_ref, bnb1_ref,
              wn2_ref, bn2_ref, wnb2_ref, bnb2_ref,
              nodes_out_ref, glob_out_ref):
    n = a_ref.shape[1]
    ii = jax.lax.broadcasted_iota(jnp.int32, (n, n), 0)
    jj = jax.lax.broadcasted_iota(jnp.int32, (n, n), 1)
    t_lt = (jj < ii).astype(jnp.bfloat16)
    t_le = (ii <= jj).astype(jnp.bfloat16)
    ones_col = jnp.ones((n, 1), jnp.bfloat16)

    layers = ((wn0_ref, bn0_ref, wnb0_ref, bnb0_ref),
              (wn1_ref, bn1_ref, wnb1_ref, bnb1_ref),
              (wn2_ref, bn2_ref, wnb2_ref, bnb2_ref))
    w_splits = [(_split(wn[...]), _split(wnb[...])) for (wn, _, wnb, _) in layers]
    fdot_t = functools.partial(jax.lax.dot_general, dimension_numbers=_DN_T,
                               preferred_element_type=jnp.float32)

    rowcnts = [jnp.sum(a_ref[g], axis=1, keepdims=True, dtype=jnp.float32)
               for g in range(_G)]
    rowcnt_all = jnp.concatenate(rowcnts, axis=1)
    ra_hi, ra_lo = _split(rowcnt_all)
    rstart_all = _fdot(t_lt, ra_hi) + _fdot(t_lt, ra_lo)
    colcnts = [jnp.sum(a_ref[g], axis=0, keepdims=True, dtype=jnp.float32)
               for g in range(_G)]
    colcnt_all = jnp.concatenate(colcnts, axis=0)
    ca_hi, ca_lo = _split(colcnt_all)
    cum_all = _fdot(ca_hi, t_le) + _fdot(ca_lo, t_le)

    w_his, w_los, presents, xs = [], [], [], []
    for g in range(_G):
        rstart = rstart_all[:, g:g + 1]
        rend = rstart + rowcnts[g]
        cum = cum_all[g:g + 1, :]
        cumprev = cum - colcnts[g]
        w_t = jnp.maximum(jnp.minimum(rend, cum) - jnp.maximum(rstart, cumprev), 0.0)
        w_hi, w_lo = _split(w_t)
        w_his.append(w_hi)
        w_los.append(w_lo)

        colcnt_col = (
            jax.lax.dot_general(w_hi, ones_col, _DN_T,
                                preferred_element_type=jnp.float32)
            + jax.lax.dot_general(w_lo, ones_col, _DN_T,
                                  preferred_element_type=jnp.float32))
        presents.append(colcnt_col > 0.0)
        xs.append(x_ref[g])

    for i, (wn, bn, wnb, bnb) in enumerate(layers):
        (wn_hi, wn_lo), (wnb_hi, wnb_lo) = w_splits[i]
        sp = [_split(x) for x in xs]
        pools = [fdot_t(w_his[g], sp[g][0]) + fdot_t(w_his[g], sp[g][1])
                 + fdot_t(w_los[g], sp[g][0]) for g in range(_G)]
        psp = [_split(p) for p in pools]
        h_nodes = [jnp.maximum(_dot3(sp[g][0], sp[g][1], wn_hi, wn_lo, _DN_M)
                               + bn[...], 0.0) for g in range(_G)]
        h_nbs = [jnp.maximum(_dot3(psp[g][0], psp[g][1], wnb_hi, wnb_lo, _DN_M)
                             + bnb[...], 0.0) for g in range(_G)]
        if i == len(layers) - 1:
            for g in range(_G):
                glob_out_ref[g] = jnp.sum(xs[g], axis=0, keepdims=True)
        xs = [jnp.where(presents[g], h_nbs[g], h_nodes[g]) for g in range(_G)]
    for g in range(_G):
        nodes_out_ref[g] = xs[g]


def kernel(nodes, adjacent,
           W_node_0, b_node_0, W_nb_0, b_nb_0,
           W_node_1, b_node_1, W_nb_1, b_nb_1,
           W_node_2, b_node_2, W_nb_2, b_nb_2):
    B, N, D = nodes.shape
    w_spec = pl.BlockSpec((D, D), lambda b: (0, 0))
    bias_spec = pl.BlockSpec((1, D), lambda b: (0, 0))
    nodes_out, glob = pl.pallas_call(
        _gnn_body,
        grid=(B // _G,),
        in_specs=[
            pl.BlockSpec((_G, N, N), lambda b: (b, 0, 0)),
            pl.BlockSpec((_G, N, D), lambda b: (b, 0, 0)),
            w_spec, bias_spec, w_spec, bias_spec,
            w_spec, bias_spec, w_spec, bias_spec,
            w_spec, bias_spec, w_spec, bias_spec,
        ],
        out_specs=[
            pl.BlockSpec((_G, N, D), lambda b: (b, 0, 0)),
            pl.BlockSpec((_G, 1, D), lambda b: (b, 0, 0)),
        ],
        out_shape=[
            jax.ShapeDtypeStruct((B, N, D), jnp.float32),
            jax.ShapeDtypeStruct((B, 1, D), jnp.float32),
        ],
    )(adjacent, nodes,
      W_node_0, b_node_0.reshape(1, D), W_nb_0, b_nb_0.reshape(1, D),
      W_node_1, b_node_1.reshape(1, D), W_nb_1, b_nb_1.reshape(1, D),
      W_node_2, b_node_2.reshape(1, D), W_nb_2, b_nb_2.reshape(1, D))
    return (nodes_out, glob.reshape(B, D))


# speedup vs baseline: 1.2377x; 1.0540x over previous
import functools

import jax
import jax.numpy as jnp
from jax.experimental import pallas as pl

_DN_T = (((0,), (0,)), ((), ()))
_DN_M = (((1,), (0,)), ((), ()))


def _split(v):
    hi = v.astype(jnp.bfloat16)
    lo = (v - hi.astype(jnp.float32)).astype(jnp.bfloat16)
    return hi, lo


def _dot3(a_hi, a_lo, b_hi, b_lo, dn):
    f = functools.partial(jax.lax.dot_general, dimension_numbers=dn,
                          preferred_element_type=jnp.float32)
    return f(a_hi, b_hi) + f(a_hi, b_lo) + f(a_lo, b_hi)


def _fdot(a, b):
    return jax.lax.dot_general(a, b, _DN_M, preferred_element_type=jnp.float32)


_G = 4


def _gnn_body(a_ref, x_ref,
              wn0_ref, bn0_ref, wnb0_ref, bnb0_ref,
              wn1_ref, bn1_ref, wnb1_ref, bnb1_ref,
              wn2_ref, bn2_ref, wnb2_ref, bnb2_ref,
              nodes_out_ref, glob_out_ref):
    n = a_ref.shape[1]
    ii = jax.lax.broadcasted_iota(jnp.int32, (n, n), 0)
    jj = jax.lax.broadcasted_iota(jnp.int32, (n, n), 1)
    t_lt = (jj < ii).astype(jnp.bfloat16)
    t_le = (ii <= jj).astype(jnp.bfloat16)
    ones_col = jnp.ones((n, 1), jnp.bfloat16)

    layers = ((wn0_ref, bn0_ref, wnb0_ref, bnb0_ref),
              (wn1_ref, bn1_ref, wnb1_ref, bnb1_ref),
              (wn2_ref, bn2_ref, wnb2_ref, bnb2_ref))
    w_splits = [(_split(wn[...]), _split(wnb[...])) for (wn, _, wnb, _) in layers]
    fdot_t = functools.partial(jax.lax.dot_general, dimension_numbers=_DN_T,
                               preferred_element_type=jnp.float32)

    w_his, w_los, presents, xs = [], [], [], []
    for g in range(_G):
        a_i = a_ref[g]
        a_bf = a_i.astype(jnp.bfloat16)

        rowcnt = jnp.sum(a_i, axis=1, keepdims=True).astype(jnp.float32)
        rc_hi, rc_lo = _split(rowcnt)
        rstart = _fdot(t_lt, rc_hi) + _fdot(t_lt, rc_lo)
        rend = rstart + rowcnt
        colcnt = jnp.sum(a_i, axis=0, keepdims=True).astype(jnp.float32)
        cc_hi, cc_lo = _split(colcnt)
        cum = _fdot(cc_hi, t_le) + _fdot(cc_lo, t_le)
        cumprev = cum - colcnt
        w_t = jnp.maximum(jnp.minimum(rend, cum) - jnp.maximum(rstart, cumprev), 0.0)
        w_hi, w_lo = _split(w_t)
        w_his.append(w_hi)
        w_los.append(w_lo)

        colcnt_col = jax.lax.dot_general(
            a_bf, ones_col, _DN_T, preferred_element_type=jnp.float32)
        presents.append(colcnt_col > 0.0)
        xs.append(x_ref[g])

    for i, (wn, bn, wnb, bnb) in enumerate(layers):
        (wn_hi, wn_lo), (wnb_hi, wnb_lo) = w_splits[i]
        sp = [_split(x) for x in xs]
        pools = [fdot_t(w_his[g], sp[g][0]) + fdot_t(w_his[g], sp[g][1])
                 + fdot_t(w_los[g], sp[g][0]) for g in range(_G)]
        psp = [_split(p) for p in pools]
        h_nodes = [jnp.maximum(_dot3(sp[g][0], sp[g][1], wn_hi, wn_lo, _DN_M)
                               + bn[...], 0.0) for g in range(_G)]
        h_nbs = [jnp.maximum(_dot3(psp[g][0], psp[g][1], wnb_hi, wnb_lo, _DN_M)
                             + bnb[...], 0.0) for g in range(_G)]
        if i == len(layers) - 1:
            for g in range(_G):
                glob_out_ref[g] = jnp.sum(xs[g], axis=0, keepdims=True)
        xs = [jnp.where(presents[g], h_nbs[g], h_nodes[g]) for g in range(_G)]
    for g in range(_G):
        nodes_out_ref[g] = xs[g]


def kernel(nodes, adjacent,
           W_node_0, b_node_0, W_nb_0, b_nb_0,
           W_node_1, b_node_1, W_nb_1, b_nb_1,
           W_node_2, b_node_2, W_nb_2, b_nb_2):
    B, N, D = nodes.shape
    w_spec = pl.BlockSpec((D, D), lambda b: (0, 0))
    bias_spec = pl.BlockSpec((1, D), lambda b: (0, 0))
    nodes_out, glob = pl.pallas_call(
        _gnn_body,
        grid=(B // _G,),
        in_specs=[
            pl.BlockSpec((_G, N, N), lambda b: (b, 0, 0)),
            pl.BlockSpec((_G, N, D), lambda b: (b, 0, 0)),
            w_spec, bias_spec, w_spec, bias_spec,
            w_spec, bias_spec, w_spec, bias_spec,
            w_spec, bias_spec, w_spec, bias_spec,
        ],
        out_specs=[
            pl.BlockSpec((_G, N, D), lambda b: (b, 0, 0)),
            pl.BlockSpec((_G, 1, D), lambda b: (b, 0, 0)),
        ],
        out_shape=[
            jax.ShapeDtypeStruct((B, N, D), jnp.float32),
            jax.ShapeDtypeStruct((B, 1, D), jnp.float32),
        ],
    )(adjacent, nodes,
      W_node_0, b_node_0.reshape(1, D), W_nb_0, b_nb_0.reshape(1, D),
      W_node_1, b_node_1.reshape(1, D), W_nb_1, b_nb_1.reshape(1, D),
      W_node_2, b_node_2.reshape(1, D), W_nb_2, b_nb_2.reshape(1, D))
    return (nodes_out, glob.reshape(B, D))
